# baseline (device time: 57538 ns/iter reference)
import functools

import jax
import jax.numpy as jnp
from jax import lax
from jax.experimental import pallas as pl
from jax.experimental.pallas import tpu as pltpu

M_HALF = 512
D = 1024


def kernel(partial, gamma):
    p = partial.reshape(partial.shape[1], partial.shape[2])
    g = gamma.reshape(1, D)

    def body(p_ref, g_ref, out_ref, recv_ref,
             send_sem1, recv_sem1, send_sem2, recv_sem2):
        my_x = lax.axis_index("x")
        my_y = lax.axis_index("y")

        barrier = pltpu.get_barrier_semaphore()
        pl.semaphore_signal(barrier, inc=1, device_id=(my_x, 1 - my_y),
                            device_id_type=pl.DeviceIdType.MESH)
        pl.semaphore_signal(barrier, inc=1, device_id=(1 - my_x, my_y),
                            device_id_type=pl.DeviceIdType.MESH)
        pl.semaphore_wait(barrier, 2)

        peer_start = (1 - my_y) * (2 * M_HALF) + my_x * M_HALF
        my_start = my_y * (2 * M_HALF) + my_x * M_HALF

        rdma1 = pltpu.make_async_remote_copy(
            src_ref=p_ref.at[pl.ds(peer_start, M_HALF), :],
            dst_ref=recv_ref,
            send_sem=send_sem1,
            recv_sem=recv_sem1,
            device_id=(my_x, 1 - my_y),
            device_id_type=pl.DeviceIdType.MESH,
        )
        rdma1.start()
        rdma1.wait()

        acc = p_ref[pl.ds(my_start, M_HALF), :] + recv_ref[:, :]
        ms = jnp.mean(acc * acc, axis=-1, keepdims=True)
        out_rows = acc * lax.rsqrt(ms + 1e-6) * g_ref[:, :]
        out_ref[pl.ds(my_x * M_HALF, M_HALF), :] = out_rows

        rdma2 = pltpu.make_async_remote_copy(
            src_ref=out_ref.at[pl.ds(my_x * M_HALF, M_HALF), :],
            dst_ref=out_ref.at[pl.ds(my_x * M_HALF, M_HALF), :],
            send_sem=send_sem2,
            recv_sem=recv_sem2,
            device_id=(1 - my_x, my_y),
            device_id_type=pl.DeviceIdType.MESH,
        )
        rdma2.start()
        rdma2.wait()

    return pl.pallas_call(
        body,
        out_shape=jax.ShapeDtypeStruct((2 * M_HALF, D), jnp.float32),
        in_specs=[
            pl.BlockSpec(memory_space=pltpu.VMEM),
            pl.BlockSpec(memory_space=pltpu.VMEM),
        ],
        out_specs=pl.BlockSpec(memory_space=pltpu.VMEM),
        scratch_shapes=[
            pltpu.VMEM((M_HALF, D), jnp.float32),
            pltpu.SemaphoreType.DMA,
            pltpu.SemaphoreType.DMA,
            pltpu.SemaphoreType.DMA,
            pltpu.SemaphoreType.DMA,
        ],
        compiler_params=pltpu.CompilerParams(collective_id=0),
    )(p, g)


# device time: 37768 ns/iter; 1.5235x vs baseline; 1.5235x over previous
import functools

import jax
import jax.numpy as jnp
from jax import lax
from jax.experimental import pallas as pl
from jax.experimental.pallas import tpu as pltpu

M_HALF = 512
D = 1024
K = 8
C = M_HALF // K


def kernel(partial, gamma):
    p = partial.reshape(partial.shape[1], partial.shape[2])
    g = gamma.reshape(1, D)

    def body(p_ref, g_ref, out_ref, recv_ref,
             send_sems1, recv_sems1, send_sems2, recv_sems2):
        my_x = lax.axis_index("x")
        my_y = lax.axis_index("y")

        barrier = pltpu.get_barrier_semaphore()
        pl.semaphore_signal(barrier, inc=1, device_id=(my_x, 1 - my_y),
                            device_id_type=pl.DeviceIdType.MESH)
        pl.semaphore_signal(barrier, inc=1, device_id=(1 - my_x, my_y),
                            device_id_type=pl.DeviceIdType.MESH)
        pl.semaphore_wait(barrier, 2)

        peer_start = (1 - my_y) * (2 * M_HALF) + my_x * M_HALF
        my_start = my_y * (2 * M_HALF) + my_x * M_HALF

        phase1 = []
        for k in range(K):
            rdma = pltpu.make_async_remote_copy(
                src_ref=p_ref.at[pl.ds(peer_start + k * C, C), :],
                dst_ref=recv_ref.at[pl.ds(k * C, C), :],
                send_sem=send_sems1.at[k],
                recv_sem=recv_sems1.at[k],
                device_id=(my_x, 1 - my_y),
                device_id_type=pl.DeviceIdType.MESH,
            )
            rdma.start()
            phase1.append(rdma)

        phase2 = []
        for k in range(K):
            phase1[k].wait_recv()
            acc = (p_ref[pl.ds(my_start + k * C, C), :]
                   + recv_ref[pl.ds(k * C, C), :])
            ms = jnp.mean(acc * acc, axis=-1, keepdims=True)
            out_rows = acc * lax.rsqrt(ms + 1e-6) * g_ref[:, :]
            out_ref[pl.ds(my_x * M_HALF + k * C, C), :] = out_rows

            rdma = pltpu.make_async_remote_copy(
                src_ref=out_ref.at[pl.ds(my_x * M_HALF + k * C, C), :],
                dst_ref=out_ref.at[pl.ds(my_x * M_HALF + k * C, C), :],
                send_sem=send_sems2.at[k],
                recv_sem=recv_sems2.at[k],
                device_id=(1 - my_x, my_y),
                device_id_type=pl.DeviceIdType.MESH,
            )
            rdma.start()
            phase2.append(rdma)

        for k in range(K):
            phase2[k].wait()
            phase1[k].wait_send()

    return pl.pallas_call(
        body,
        out_shape=jax.ShapeDtypeStruct((2 * M_HALF, D), jnp.float32),
        in_specs=[
            pl.BlockSpec(memory_space=pltpu.VMEM),
            pl.BlockSpec(memory_space=pltpu.VMEM),
        ],
        out_specs=pl.BlockSpec(memory_space=pltpu.VMEM),
        scratch_shapes=[
            pltpu.VMEM((M_HALF, D), jnp.float32),
            pltpu.SemaphoreType.DMA((K,)),
            pltpu.SemaphoreType.DMA((K,)),
            pltpu.SemaphoreType.DMA((K,)),
            pltpu.SemaphoreType.DMA((K,)),
        ],
        compiler_params=pltpu.CompilerParams(collective_id=0),
    )(p, g)
